# baseline (speedup 1.0000x reference)
import jax
import jax.numpy as jnp
from jax.experimental import pallas as pl

MIN_VAL = -8.0
MAX_VAL = 8.0
SCALE = (MAX_VAL - MIN_VAL) / 15.0


def _quant_body(x_ref, o_ref):
    x = x_ref[...]
    c = jnp.clip(jnp.round(x), MIN_VAL, MAX_VAL)
    q = jnp.round((c - MIN_VAL) / SCALE)
    o_ref[...] = q * SCALE + MIN_VAL


def kernel(x):
    rows, cols = x.shape
    block_rows = 128
    grid = (rows // block_rows,)
    return pl.pallas_call(
        _quant_body,
        grid=grid,
        in_specs=[pl.BlockSpec((block_rows, cols), lambda i: (i, 0))],
        out_specs=pl.BlockSpec((block_rows, cols), lambda i: (i, 0)),
        out_shape=jax.ShapeDtypeStruct((rows, cols), jnp.float32),
    )(x)
